# baseline (speedup 1.0000x reference)
import functools

import jax
import jax.numpy as jnp
from jax import lax
from jax.experimental import pallas as pl
from jax.experimental.pallas import tpu as pltpu
from jax.experimental.pallas import tpu_sc as plsc

B, H, W = 16, 512, 512
NPTS = 256
NW = 32
P = 2 * B * NPTS
PPW = P // NW
LANES = 16
NBKT = 4
CAP = 128
BPAD = 160


def _sc_gather(mask2d, xs, ys):
    mesh = plsc.VectorSubcoreMesh(core_axis_name="c", subcore_axis_name="s")

    @functools.partial(
        pl.kernel,
        out_type=jax.ShapeDtypeStruct((P,), jnp.float32),
        mesh=mesh,
        scratch_types=[
            pltpu.VMEM((PPW,), jnp.int32),
            pltpu.VMEM((PPW,), jnp.int32),
            pltpu.VMEM((NBKT, BPAD), jnp.int32),
            pltpu.VMEM((NBKT, BPAD), jnp.int32),
            pltpu.VMEM((CAP, 128), jnp.float32),
            pltpu.VMEM((CAP, 128), jnp.float32),
            pltpu.VMEM((CAP, 128), jnp.float32),
            pltpu.VMEM((CAP, 128), jnp.float32),
            pltpu.VMEM((PPW,), jnp.float32),
            pltpu.SemaphoreType.DMA,
            pltpu.SemaphoreType.DMA,
            pltpu.SemaphoreType.DMA,
            pltpu.SemaphoreType.DMA,
        ],
        compiler_params=pltpu.CompilerParams(
            use_tc_tiling_on_sc=True, needs_layout_passes=False
        ),
    )
    def k(mask_hbm, xs_hbm, ys_hbm, out_hbm, xv, yv, brows, baux,
          buf0, buf1, buf2, buf3, valv, s0, s1, s2, s3):
        wid = lax.axis_index("s") * 2 + lax.axis_index("c")
        base = wid * PPW
        brow = (wid % B) * H
        pltpu.sync_copy(xs_hbm.at[pl.ds(base, PPW)], xv)
        pltpu.sync_copy(ys_hbm.at[pl.ds(base, PPW)], yv)
        iota = lax.iota(jnp.int32, LANES)
        zero16 = iota * 0

        def clear_body(j, _):
            brows[j // (BPAD // LANES),
                  pl.ds((j % (BPAD // LANES)) * LANES, LANES)] = zero16
            return 0

        lax.fori_loop(0, NBKT * (BPAD // LANES), clear_body, 0)

        def part_body(j, offs):
            sl = pl.ds(j * LANES, LANES)
            x = xv[sl]
            row = yv[sl] + brow
            aux = (j * LANES + iota) * 128 + (x & 127)
            xt = x >> 7
            new_offs = []
            for t in range(NBKT):
                m = xt == t
                rank = plsc.cumsum(m.astype(jnp.int32)) - 1
                dest = offs[t] + rank
                tvec = zero16 + t
                plsc.store_scatter(brows, [tvec, dest], row, mask=m)
                plsc.store_scatter(baux, [tvec, dest], aux, mask=m)
                new_offs.append(
                    offs[t] + plsc.all_reduce_population_count(m)
                )
            return tuple(new_offs)

        offs = lax.fori_loop(
            0, PPW // LANES, part_body,
            (zero16, zero16, zero16, zero16),
        )

        bufs = (buf0, buf1, buf2, buf3)
        sems = (s0, s1, s2, s3)
        handles = []
        for t in range(NBKT):
            handles.append(
                pltpu.async_copy(
                    mask_hbm.at[brows.at[t, pl.ds(0, CAP)],
                                pl.ds(t * 128, 128)],
                    bufs[t],
                    sems[t],
                )
            )
        for t in range(NBKT):
            handles[t].wait()
            cnt = offs[t]

            def ex_body(kk, _, t=t, cnt=cnt):
                lane = kk * LANES + iota
                valid = lane < cnt
                aux = baux[t, pl.ds(kk * LANES, LANES)]
                col = aux & 127
                pos = aux >> 7
                vals = plsc.load_gather(bufs[t], [lane, col])
                plsc.store_scatter(valv, [pos], vals, mask=valid)
                return 0

            lax.fori_loop(0, CAP // LANES, ex_body, 0)
        pltpu.sync_copy(valv, out_hbm.at[pl.ds(base, PPW)])

    return k(mask2d, xs, ys)


def _tc_reduce_body(v_ref, o_ref):
    v = v_ref[:]
    t = (lax.broadcasted_iota(jnp.int32, v.shape, 0) < 32).astype(jnp.float32)
    loss = jnp.maximum(v, 0.0) - v * t + jnp.log1p(jnp.exp(-jnp.abs(v)))
    o_ref[0, 0] = jnp.sum(loss) * (1.0 / P)


def _tc_reduce(vals):
    return pl.pallas_call(
        _tc_reduce_body,
        out_shape=jax.ShapeDtypeStruct((1, 1), jnp.float32),
        in_specs=[pl.BlockSpec(memory_space=pltpu.VMEM)],
        out_specs=pl.BlockSpec(memory_space=pltpu.SMEM),
    )(vals)


def kernel(pred_mask, positive_points, negative_points):
    pp = positive_points.astype(jnp.int32)
    np_ = negative_points.astype(jnp.int32)
    xs = jnp.concatenate([pp[:, :, 0].reshape(-1), np_[:, :, 0].reshape(-1)])
    ys = jnp.concatenate([pp[:, :, 1].reshape(-1), np_[:, :, 1].reshape(-1)])
    vals = _sc_gather(pred_mask.reshape(B * H, W), xs, ys)
    return _tc_reduce(vals.reshape(P // 128, 128))[0, 0]


# speedup vs baseline: 6.4097x; 6.4097x over previous
import functools

import jax
import jax.numpy as jnp
from jax import lax
from jax.experimental import pallas as pl
from jax.experimental.pallas import tpu as pltpu
from jax.experimental.pallas import tpu_sc as plsc

B, H, W = 16, 512, 512
NPTS = 256
NW = 32
P = 2 * B * NPTS
PPW = P // NW
LANES = 16


def _sc_gather(mask2d, px, py, nx, ny):
    mesh = plsc.VectorSubcoreMesh(core_axis_name="c", subcore_axis_name="s")
    NQ = 4
    QR = PPW // NQ

    @functools.partial(
        pl.kernel,
        out_type=jax.ShapeDtypeStruct((P,), jnp.float32),
        mesh=mesh,
        scratch_types=[
            pltpu.VMEM((PPW,), jnp.int32),
            pltpu.VMEM((PPW,), jnp.int32),
            pltpu.VMEM((PPW,), jnp.int32),
            pltpu.VMEM((QR, W), jnp.float32),
            pltpu.VMEM((QR, W), jnp.float32),
            pltpu.VMEM((PPW,), jnp.float32),
            pltpu.SemaphoreType.DMA,
            pltpu.SemaphoreType.DMA,
        ],
        compiler_params=pltpu.CompilerParams(
            use_tc_tiling_on_sc=True, needs_layout_passes=False
        ),
    )
    def k(mask_hbm, px_hbm, py_hbm, nx_hbm, ny_hbm, out_hbm,
          xv, yv, idxv, rows0, rows1, valv, sem0, sem1):
        wid = lax.axis_index("s") * 2 + lax.axis_index("c")
        base = wid * PPW
        b = wid % B
        brow = b * H
        bpt = b * NPTS

        @pl.when(wid < B)
        def _():
            pltpu.sync_copy(px_hbm.at[pl.ds(bpt, PPW)], xv)
            pltpu.sync_copy(py_hbm.at[pl.ds(bpt, PPW)], yv)

        @pl.when(wid >= B)
        def _():
            pltpu.sync_copy(nx_hbm.at[pl.ds(bpt, PPW)], xv)
            pltpu.sync_copy(ny_hbm.at[pl.ds(bpt, PPW)], yv)

        def idx_body(j, _):
            sl = pl.ds(j * LANES, LANES)
            idxv[sl] = yv[sl] + brow
            return 0

        lax.fori_loop(0, PPW // LANES, idx_body, 0)

        bufs = (rows0, rows1)
        sems = (sem0, sem1)

        def fire(q):
            return pltpu.async_copy(
                mask_hbm.at[idxv.at[pl.ds(q * QR, QR)]], bufs[q % 2],
                sems[q % 2],
            )
        handles = [fire(0), fire(1)] + [None] * (NQ - 2)
        iota = lax.iota(jnp.int32, LANES)
        for q in range(NQ):
            handles[q].wait()
            buf = bufs[q % 2]
            qb = q * QR

            def ex_body(j, _, buf=buf, qb=qb):
                sl = pl.ds(qb + j * LANES, LANES)
                valv[sl] = plsc.load_gather(buf, [j * LANES + iota, xv[sl]])
                return 0

            lax.fori_loop(0, QR // LANES, ex_body, 0)
            if q + 2 < NQ:
                handles[q + 2] = fire(q + 2)
        pltpu.sync_copy(valv, out_hbm.at[pl.ds(base, PPW)])

    return k(mask2d, px, py, nx, ny)


def _tc_reduce_body(v_ref, o_ref):
    v = v_ref[:]
    t = (lax.broadcasted_iota(jnp.int32, v.shape, 0) < 32).astype(jnp.float32)
    loss = jnp.maximum(v, 0.0) - v * t + jnp.log1p(jnp.exp(-jnp.abs(v)))
    o_ref[0, 0] = jnp.sum(loss) * (1.0 / P)


def _tc_reduce(vals):
    return pl.pallas_call(
        _tc_reduce_body,
        out_shape=jax.ShapeDtypeStruct((1, 1), jnp.float32),
        in_specs=[pl.BlockSpec(memory_space=pltpu.VMEM)],
        out_specs=pl.BlockSpec(memory_space=pltpu.SMEM),
    )(vals)


def kernel(pred_mask, positive_points, negative_points):
    pp = positive_points.astype(jnp.int32)
    np_ = negative_points.astype(jnp.int32)
    vals = _sc_gather(
        pred_mask.reshape(B * H, W),
        pp[:, :, 0].reshape(-1),
        pp[:, :, 1].reshape(-1),
        np_[:, :, 0].reshape(-1),
        np_[:, :, 1].reshape(-1),
    )
    return _tc_reduce(vals.reshape(P // 128, 128))[0, 0]
